# Initial kernel scaffold; baseline (speedup 1.0000x reference)
import jax
import jax.numpy as jnp
from jax.experimental import pallas as pl


def kernel(hidden_states, Wg, W_gate, W_up, W_down):
    raise NotImplementedError("write your pallas kernel here")


# speedup vs baseline: 4.1570x; 4.1570x over previous
import functools

import jax
import jax.numpy as jnp
from jax import lax
from jax.experimental import pallas as pl
from jax.experimental.pallas import tpu as pltpu
from jax.experimental.pallas import tpu_sc as plsc

_TOPK = 2
_BM = 64
_NC = 2
_NS = 16
_NW = _NC * _NS


def _router_body(x_ref, wg_ref, out_ref):
    out_ref[...] = jnp.dot(x_ref[...], wg_ref[...],
                           preferred_element_type=jnp.float32)


def _ffn_body(be_ref, nb_ref, x_ref, wg_ref, wu_ref, wd_ref, y_ref):
    del be_ref

    @pl.when(pl.program_id(0) < nb_ref[0])
    def _():
        x = x_ref[...]
        g = jnp.dot(x, wg_ref[0], preferred_element_type=jnp.float32)
        u = jnp.dot(x, wu_ref[0], preferred_element_type=jnp.float32)
        h = (g * jax.nn.sigmoid(g)) * u
        y_ref[...] = jnp.dot(h, wd_ref[0], preferred_element_type=jnp.float32)


def _combine_body(y0_ref, y1_ref, w0_ref, w1_ref, out_ref):
    out_ref[...] = w0_ref[...] * y0_ref[...] + w1_ref[...] * y1_ref[...]


def kernel(hidden_states, Wg, W_gate, W_up, W_down):
    b, s, d = hidden_states.shape
    e, _, dff = W_gate.shape
    t = b * s
    x = hidden_states.reshape(t, d)

    rb = 256
    router_logits = pl.pallas_call(
        _router_body,
        grid=(t // rb,),
        in_specs=[
            pl.BlockSpec((rb, d), lambda i: (i, 0)),
            pl.BlockSpec((d, e), lambda i: (0, 0)),
        ],
        out_specs=pl.BlockSpec((rb, e), lambda i: (i, 0)),
        out_shape=jax.ShapeDtypeStruct((t, e), jnp.float32),
    )(x, Wg)

    l1 = router_logits
    a1 = jnp.argmax(l1, axis=-1).astype(jnp.int32)
    m1 = jnp.max(l1, axis=-1)
    l2 = jnp.where(jax.nn.one_hot(a1, e, dtype=jnp.bool_), -jnp.inf, l1)
    a2 = jnp.argmax(l2, axis=-1).astype(jnp.int32)
    m2 = jnp.max(l2, axis=-1)
    w1 = jax.nn.sigmoid(m1 - m2)
    w2 = 1.0 - w1
    sel_flat = jnp.stack([a1, a2], axis=-1).reshape(-1)

    nassign = t * _TOPK
    nblk_max = nassign // _BM + e
    rpad = nblk_max * _BM
    oh = jax.nn.one_hot(sel_flat, e, dtype=jnp.int32)
    cum = jnp.cumsum(oh, axis=0)
    counts = cum[-1]
    rank = jnp.take_along_axis(cum, sel_flat[:, None], axis=1)[:, 0] - 1
    nblk = (counts + _BM - 1) // _BM
    cnb = jnp.cumsum(nblk)
    base = (cnb - nblk) * _BM
    pad_slot = (base[sel_flat] + rank).astype(jnp.int32)
    nb_total = cnb[-1].astype(jnp.int32)
    blk_ids = jnp.arange(nblk_max, dtype=jnp.int32)
    block_expert = jnp.searchsorted(cnb, blk_ids, side='right').astype(jnp.int32)
    last_e = jnp.searchsorted(cnb, nb_total - 1, side='right').astype(jnp.int32)
    block_expert = jnp.where(blk_ids < nb_total, block_expert, last_e)
    tok_flat = jnp.arange(nassign, dtype=jnp.int32) // _TOPK

    a_per_w = nassign // _NW
    mesh = plsc.VectorSubcoreMesh(core_axis_name="c", subcore_axis_name="s")

    @functools.partial(
        pl.kernel, mesh=mesh,
        out_type=jax.ShapeDtypeStruct((rpad, d), jnp.float32),
        scratch_types=[
            pltpu.VMEM((a_per_w,), jnp.int32),
            pltpu.VMEM((a_per_w,), jnp.int32),
            pltpu.VMEM((a_per_w, d), jnp.float32),
            pltpu.SemaphoreType.DMA,
        ],
    )
    def _dispatch(x_hbm, tok_hbm, slot_hbm, xd_hbm, tok_v, slot_v, rows_v, sem):
        wid = lax.axis_index("s") * _NC + lax.axis_index("c")
        off = wid * a_per_w
        pltpu.sync_copy(tok_hbm.at[pl.ds(off, a_per_w)], tok_v)
        pltpu.sync_copy(slot_hbm.at[pl.ds(off, a_per_w)], slot_v)
        pltpu.async_copy(x_hbm.at[tok_v], rows_v, sem).wait()
        pltpu.async_copy(rows_v, xd_hbm.at[slot_v], sem).wait()

    x_disp = _dispatch(x, tok_flat, pad_slot)

    grid_spec = pltpu.PrefetchScalarGridSpec(
        num_scalar_prefetch=2,
        grid=(nblk_max,),
        in_specs=[
            pl.BlockSpec((_BM, d), lambda i, be, nb: (i, 0)),
            pl.BlockSpec((1, d, dff), lambda i, be, nb: (be[i], 0, 0)),
            pl.BlockSpec((1, d, dff), lambda i, be, nb: (be[i], 0, 0)),
            pl.BlockSpec((1, dff, d), lambda i, be, nb: (be[i], 0, 0)),
        ],
        out_specs=pl.BlockSpec((_BM, d), lambda i, be, nb: (i, 0)),
    )
    y = pl.pallas_call(
        _ffn_body,
        grid_spec=grid_spec,
        out_shape=jax.ShapeDtypeStruct((rpad, d), jnp.float32),
    )(block_expert, nb_total.reshape(1), x_disp, W_gate, W_up, W_down)

    p = pad_slot.reshape(t, _TOPK)
    p0 = p[:, 0]
    p1 = p[:, 1]
    t_per_w = t // _NW

    @functools.partial(
        pl.kernel, mesh=mesh,
        out_type=[jax.ShapeDtypeStruct((t, d), jnp.float32),
                  jax.ShapeDtypeStruct((t, d), jnp.float32)],
        scratch_types=[
            pltpu.VMEM((t_per_w,), jnp.int32),
            pltpu.VMEM((t_per_w, d), jnp.float32),
            pltpu.SemaphoreType.DMA,
        ],
    )
    def _cgather(y_hbm, p0_hbm, p1_hbm, y0_hbm, y1_hbm, idx_v, rows_v, sem):
        wid = lax.axis_index("s") * _NC + lax.axis_index("c")
        off = wid * t_per_w
        pltpu.sync_copy(p0_hbm.at[pl.ds(off, t_per_w)], idx_v)
        pltpu.async_copy(y_hbm.at[idx_v], rows_v, sem).wait()
        pltpu.sync_copy(rows_v, y0_hbm.at[pl.ds(off, t_per_w)])
        pltpu.sync_copy(p1_hbm.at[pl.ds(off, t_per_w)], idx_v)
        pltpu.async_copy(y_hbm.at[idx_v], rows_v, sem).wait()
        pltpu.sync_copy(rows_v, y1_hbm.at[pl.ds(off, t_per_w)])

    yg0, yg1 = _cgather(y, p0, p1)

    out = pl.pallas_call(
        _combine_body,
        grid=(t // rb,),
        in_specs=[
            pl.BlockSpec((rb, d), lambda i: (i, 0)),
            pl.BlockSpec((rb, d), lambda i: (i, 0)),
            pl.BlockSpec((rb, 1), lambda i: (i, 0)),
            pl.BlockSpec((rb, 1), lambda i: (i, 0)),
        ],
        out_specs=pl.BlockSpec((rb, d), lambda i: (i, 0)),
        out_shape=jax.ShapeDtypeStruct((t, d), jnp.float32),
    )(yg0, yg1, w1.reshape(t, 1), w2.reshape(t, 1))

    return out.reshape(b, s, d), router_logits


# speedup vs baseline: 7.6265x; 1.8346x over previous
import functools

import jax
import jax.numpy as jnp
from jax import lax
from jax.experimental import pallas as pl
from jax.experimental.pallas import tpu as pltpu
from jax.experimental.pallas import tpu_sc as plsc

_TOPK = 2
_BM = 128
_RB = 256
_NC = 2
_NS = 16
_NW = _NC * _NS


def _top2(l, e):
    rb = l.shape[0]
    lane = jax.lax.broadcasted_iota(jnp.int32, (rb, e), 1)
    m1 = jnp.max(l, axis=1, keepdims=True)
    a1 = jnp.min(jnp.where(l == m1, lane, e), axis=1, keepdims=True)
    l2 = jnp.where(lane == a1, -jnp.inf, l)
    m2 = jnp.max(l2, axis=1, keepdims=True)
    a2 = jnp.min(jnp.where(l2 == m2, lane, e), axis=1, keepdims=True)
    return a1, m1, a2, m2


def _pack_pair(lo_bf, hi_bf):
    lo = jax.lax.bitcast_convert_type(lo_bf, jnp.uint16).astype(jnp.uint32)
    hi = jax.lax.bitcast_convert_type(hi_bf, jnp.uint16).astype(jnp.uint32)
    return jax.lax.bitcast_convert_type(lo | (hi << 16), jnp.int32)


def _unpack_pair(w_i32):
    w = jax.lax.bitcast_convert_type(w_i32, jnp.uint32)
    lo = jax.lax.bitcast_convert_type((w & 0xFFFF).astype(jnp.uint16),
                                      jnp.bfloat16)
    hi = jax.lax.bitcast_convert_type((w >> 16).astype(jnp.uint16),
                                      jnp.bfloat16)
    return jnp.concatenate([lo, hi], axis=1)


def _make_routing_body(e, nblk_max):
    def body(x_ref, wg_ref, l_ref, p0_ref, p1_ref, w1_ref, w2_ref, be_ref,
             nb_ref, c0r_ref, c1r_ref, c0f_ref, base_ref, lsc_ref):
        ph = pl.program_id(0)
        i = pl.program_id(1)
        rb = x_ref.shape[0]

        @pl.when(ph == 0)
        def _():
            lsc_ref[pl.ds(i * rb, rb), :] = jnp.dot(
                x_ref[...], wg_ref[...], preferred_element_type=jnp.float32)

        l = lsc_ref[pl.ds(i * rb, rb), :]
        l_ref[...] = l
        a1, m1, a2, m2 = _top2(l, e)
        lane = jax.lax.broadcasted_iota(jnp.int32, (_RB, e), 1)
        oh1 = (lane == a1)
        oh2 = (lane == a2)
        s1 = jnp.sum(oh1.astype(jnp.float32), axis=0, keepdims=True)
        s2 = jnp.sum(oh2.astype(jnp.float32), axis=0, keepdims=True)

        @pl.when((ph == 0) & (i == 0))
        def _():
            c0r_ref[...] = jnp.zeros_like(c0r_ref)
            c1r_ref[...] = jnp.zeros_like(c1r_ref)

        @pl.when(ph == 0)
        def _():
            c0r_ref[...] += s1
            c1r_ref[...] += s2

        @pl.when((ph == 1) & (i == 0))
        def _():
            c0 = c0r_ref[...]
            counts = c0 + c1r_ref[...]
            c0f_ref[...] = c0
            tril_e = (jax.lax.broadcasted_iota(jnp.int32, (e, e), 0)
                      <= jax.lax.broadcasted_iota(jnp.int32, (e, e), 1)
                      ).astype(jnp.float32)
            nblk = jnp.ceil(counts / _BM)
            cnb = jnp.dot(nblk, tril_e, preferred_element_type=jnp.float32)
            base_ref[...] = (cnb - nblk) * _BM
            nb_total = cnb[0, e - 1]
            nb_ref[0] = nb_total.astype(jnp.int32)
            cnb_b = jnp.broadcast_to(cnb, (nblk_max, e))
            jrow = jax.lax.broadcasted_iota(
                jnp.int32, (nblk_max, e), 0).astype(jnp.float32)
            be = jnp.sum((cnb_b <= jrow).astype(jnp.int32), axis=1)
            be_last = jnp.sum((cnb_b <= (nb_total - 1.0)).astype(jnp.int32),
                              axis=1)
            be_ref[...] = jnp.minimum(be, be_last)
            c0r_ref[...] = jnp.zeros_like(c0r_ref)
            c1r_ref[...] = jnp.zeros_like(c1r_ref)

        @pl.when(ph == 1)
        def _():
            tril = (jax.lax.broadcasted_iota(jnp.int32, (_RB, _RB), 0)
                    >= jax.lax.broadcasted_iota(jnp.int32, (_RB, _RB), 1)
                    ).astype(jnp.bfloat16)
            cum1 = jnp.dot(tril, oh1.astype(jnp.bfloat16),
                           preferred_element_type=jnp.float32) + c0r_ref[...]
            cum2 = jnp.dot(tril, oh2.astype(jnp.bfloat16),
                           preferred_element_type=jnp.float32) + c1r_ref[...]
            base = base_ref[...]
            c0f = c0f_ref[...]
            p0 = jnp.sum(jnp.where(oh1, base + cum1 - 1.0, 0.0), axis=1)
            p1 = jnp.sum(jnp.where(oh2, base + c0f + cum2 - 1.0, 0.0), axis=1)
            p0_ref[...] = p0.astype(jnp.int32)
            p1_ref[...] = p1.astype(jnp.int32)
            w1v = jax.nn.sigmoid(m1 - m2)[:, 0]
            w1_ref[...] = w1v
            w2_ref[...] = 1.0 - w1v
            c0r_ref[...] += s1
            c1r_ref[...] += s2

    return body


def _ffn_body(be_ref, nb_ref, x_ref, wg_ref, wu_ref, wd_ref, y_ref):
    del be_ref

    @pl.when(pl.program_id(0) < nb_ref[0])
    def _():
        x = _unpack_pair(x_ref[...])
        g = jnp.dot(x, wg_ref[0].astype(jnp.bfloat16),
                    preferred_element_type=jnp.float32)
        u = jnp.dot(x, wu_ref[0].astype(jnp.bfloat16),
                    preferred_element_type=jnp.float32)
        h = (g * jax.nn.sigmoid(g)) * u
        y = jnp.dot(h.astype(jnp.bfloat16), wd_ref[0].astype(jnp.bfloat16),
                    preferred_element_type=jnp.float32)
        dh = y.shape[1] // 2
        y_ref[...] = _pack_pair(y[:, :dh].astype(jnp.bfloat16),
                                y[:, dh:].astype(jnp.bfloat16))


def _combine_body(y0_ref, y1_ref, w0_ref, w1_ref, out_ref):
    y0 = _unpack_pair(y0_ref[...]).astype(jnp.float32)
    y1 = _unpack_pair(y1_ref[...]).astype(jnp.float32)
    out_ref[...] = w0_ref[...] * y0 + w1_ref[...] * y1


def kernel(hidden_states, Wg, W_gate, W_up, W_down):
    b, s, d = hidden_states.shape
    e, _, dff = W_gate.shape
    t = b * s
    x = hidden_states.reshape(t, d)

    nblk_max = t * _TOPK // _BM + e
    rpad = nblk_max * _BM
    dp = d // 2

    (router_logits, p0, p1, w1, w2, block_expert, nb_total) = pl.pallas_call(
        _make_routing_body(e, nblk_max),
        grid=(2, t // _RB),
        in_specs=[
            pl.BlockSpec((_RB, d), lambda ph, i: ((1 - ph) * i, 0)),
            pl.BlockSpec((d, e), lambda ph, i: (0, 0)),
        ],
        out_specs=[
            pl.BlockSpec((_RB, e), lambda ph, i: (i, 0)),
            pl.BlockSpec((_RB,), lambda ph, i: (i,)),
            pl.BlockSpec((_RB,), lambda ph, i: (i,)),
            pl.BlockSpec((_RB,), lambda ph, i: (i,)),
            pl.BlockSpec((_RB,), lambda ph, i: (i,)),
            pl.BlockSpec((nblk_max,), lambda ph, i: (0,)),
            pl.BlockSpec(memory_space=pltpu.SMEM),
        ],
        out_shape=[
            jax.ShapeDtypeStruct((t, e), jnp.float32),
            jax.ShapeDtypeStruct((t,), jnp.int32),
            jax.ShapeDtypeStruct((t,), jnp.int32),
            jax.ShapeDtypeStruct((t,), jnp.float32),
            jax.ShapeDtypeStruct((t,), jnp.float32),
            jax.ShapeDtypeStruct((nblk_max,), jnp.int32),
            jax.ShapeDtypeStruct((1,), jnp.int32),
        ],
        scratch_shapes=[
            pltpu.VMEM((1, e), jnp.float32),
            pltpu.VMEM((1, e), jnp.float32),
            pltpu.VMEM((1, e), jnp.float32),
            pltpu.VMEM((1, e), jnp.float32),
            pltpu.VMEM((t, e), jnp.float32),
        ],
    )(x, Wg)

    t_per_w = t // _NW
    mesh = plsc.VectorSubcoreMesh(core_axis_name="c", subcore_axis_name="s")

    @functools.partial(
        pl.kernel, mesh=mesh,
        out_type=jax.ShapeDtypeStruct((rpad, dp), jnp.int32),
        scratch_types=[
            pltpu.VMEM((t_per_w,), jnp.int32),
            pltpu.VMEM((t_per_w, dp), jnp.int32),
            pltpu.SemaphoreType.DMA,
        ],
    )
    def _dispatch(x_hbm, p0_hbm, p1_hbm, xd_hbm, slot_v, rows_v, sem):
        wid = lax.axis_index("s") * _NC + lax.axis_index("c")
        off = wid * t_per_w
        pltpu.sync_copy(x_hbm.at[pl.ds(off, t_per_w)], rows_v)
        pltpu.sync_copy(p0_hbm.at[pl.ds(off, t_per_w)], slot_v)
        pltpu.async_copy(rows_v, xd_hbm.at[slot_v], sem).wait()
        pltpu.sync_copy(p1_hbm.at[pl.ds(off, t_per_w)], slot_v)
        pltpu.async_copy(rows_v, xd_hbm.at[slot_v], sem).wait()

    xb = x.astype(jnp.bfloat16)
    x_packed = _pack_pair(xb[:, :dp], xb[:, dp:])
    x_disp = _dispatch(x_packed, p0, p1)

    grid_spec = pltpu.PrefetchScalarGridSpec(
        num_scalar_prefetch=2,
        grid=(nblk_max,),
        in_specs=[
            pl.BlockSpec((_BM, dp), lambda i, be, nb: (jnp.minimum(i, nb[0]), 0)),
            pl.BlockSpec((1, d, dff), lambda i, be, nb: (be[i], 0, 0)),
            pl.BlockSpec((1, d, dff), lambda i, be, nb: (be[i], 0, 0)),
            pl.BlockSpec((1, dff, d), lambda i, be, nb: (be[i], 0, 0)),
        ],
        out_specs=pl.BlockSpec((_BM, dp), lambda i, be, nb: (jnp.minimum(i, nb[0]), 0)),
    )
    y = pl.pallas_call(
        _ffn_body,
        grid_spec=grid_spec,
        out_shape=jax.ShapeDtypeStruct((rpad, dp), jnp.int32),
    )(block_expert, nb_total, x_disp, W_gate, W_up, W_down)

    g_per_w = t // _NW

    @functools.partial(
        pl.kernel, mesh=mesh,
        out_type=[jax.ShapeDtypeStruct((t, dp), jnp.int32),
                  jax.ShapeDtypeStruct((t, dp), jnp.int32)],
        scratch_types=[
            pltpu.VMEM((g_per_w,), jnp.int32),
            pltpu.VMEM((g_per_w, dp), jnp.int32),
            pltpu.SemaphoreType.DMA,
        ],
    )
    def _cgather(y_hbm, p0_hbm, p1_hbm, y0_hbm, y1_hbm, idx_v, rows_v, sem):
        wid = lax.axis_index("s") * _NC + lax.axis_index("c")
        off = wid * g_per_w
        pltpu.sync_copy(p0_hbm.at[pl.ds(off, g_per_w)], idx_v)
        pltpu.async_copy(y_hbm.at[idx_v], rows_v, sem).wait()
        pltpu.sync_copy(rows_v, y0_hbm.at[pl.ds(off, g_per_w)])
        pltpu.sync_copy(p1_hbm.at[pl.ds(off, g_per_w)], idx_v)
        pltpu.async_copy(y_hbm.at[idx_v], rows_v, sem).wait()
        pltpu.sync_copy(rows_v, y1_hbm.at[pl.ds(off, g_per_w)])

    yg0, yg1 = _cgather(y, p0, p1)

    out = pl.pallas_call(
        _combine_body,
        grid=(t // _RB,),
        in_specs=[
            pl.BlockSpec((_RB, dp), lambda i: (i, 0)),
            pl.BlockSpec((_RB, dp), lambda i: (i, 0)),
            pl.BlockSpec((_RB, 1), lambda i: (i, 0)),
            pl.BlockSpec((_RB, 1), lambda i: (i, 0)),
        ],
        out_specs=pl.BlockSpec((_RB, d), lambda i: (i, 0)),
        out_shape=jax.ShapeDtypeStruct((t, d), jnp.float32),
    )(yg0, yg1, w1.reshape(t, 1), w2.reshape(t, 1))

    return out.reshape(b, s, d), router_logits


# speedup vs baseline: 7.7985x; 1.0225x over previous
import functools

import jax
import jax.numpy as jnp
from jax import lax
from jax.experimental import pallas as pl
from jax.experimental.pallas import tpu as pltpu
from jax.experimental.pallas import tpu_sc as plsc

_TOPK = 2
_BM = 128
_RB = 512
_NC = 2
_NS = 16
_NW = _NC * _NS


def _top2(l, e):
    rb = l.shape[0]
    lane = jax.lax.broadcasted_iota(jnp.int32, (rb, e), 1)
    m1 = jnp.max(l, axis=1, keepdims=True)
    a1 = jnp.min(jnp.where(l == m1, lane, e), axis=1, keepdims=True)
    l2 = jnp.where(lane == a1, -jnp.inf, l)
    m2 = jnp.max(l2, axis=1, keepdims=True)
    a2 = jnp.min(jnp.where(l2 == m2, lane, e), axis=1, keepdims=True)
    return a1, m1, a2, m2


def _pack_pair(lo_bf, hi_bf):
    lo = jax.lax.bitcast_convert_type(lo_bf, jnp.uint16).astype(jnp.uint32)
    hi = jax.lax.bitcast_convert_type(hi_bf, jnp.uint16).astype(jnp.uint32)
    return jax.lax.bitcast_convert_type(lo | (hi << 16), jnp.int32)


def _unpack_pair(w_i32):
    w = jax.lax.bitcast_convert_type(w_i32, jnp.uint32)
    lo = jax.lax.bitcast_convert_type((w & 0xFFFF).astype(jnp.uint16),
                                      jnp.bfloat16)
    hi = jax.lax.bitcast_convert_type((w >> 16).astype(jnp.uint16),
                                      jnp.bfloat16)
    return jnp.concatenate([lo, hi], axis=1)


def _make_routing_body(e, nblk_max):
    def body(x_ref, wg_ref, l_ref, p0_ref, p1_ref, w1_ref, w2_ref, be_ref,
             nb_ref, c0r_ref, c1r_ref, c0f_ref, base_ref, lsc_ref):
        ph = pl.program_id(0)
        i = pl.program_id(1)
        rb = x_ref.shape[0]

        @pl.when(ph == 0)
        def _():
            lsc_ref[pl.ds(i * rb, rb), :] = jnp.dot(
                x_ref[...], wg_ref[...], preferred_element_type=jnp.float32)

        l = lsc_ref[pl.ds(i * rb, rb), :]
        l_ref[...] = l
        a1, m1, a2, m2 = _top2(l, e)
        lane = jax.lax.broadcasted_iota(jnp.int32, (_RB, e), 1)
        oh1 = (lane == a1)
        oh2 = (lane == a2)
        s1 = jnp.sum(oh1.astype(jnp.float32), axis=0, keepdims=True)
        s2 = jnp.sum(oh2.astype(jnp.float32), axis=0, keepdims=True)

        @pl.when((ph == 0) & (i == 0))
        def _():
            c0r_ref[...] = jnp.zeros_like(c0r_ref)
            c1r_ref[...] = jnp.zeros_like(c1r_ref)

        @pl.when(ph == 0)
        def _():
            c0r_ref[...] += s1
            c1r_ref[...] += s2

        @pl.when((ph == 1) & (i == 0))
        def _():
            c0 = c0r_ref[...]
            counts = c0 + c1r_ref[...]
            c0f_ref[...] = c0
            tril_e = (jax.lax.broadcasted_iota(jnp.int32, (e, e), 0)
                      <= jax.lax.broadcasted_iota(jnp.int32, (e, e), 1)
                      ).astype(jnp.float32)
            nblk = jnp.ceil(counts / _BM)
            cnb = jnp.dot(nblk, tril_e, preferred_element_type=jnp.float32)
            base_ref[...] = (cnb - nblk) * _BM
            nb_total = cnb[0, e - 1]
            nb_ref[0] = nb_total.astype(jnp.int32)
            cnb_b = jnp.broadcast_to(cnb, (nblk_max, e))
            jrow = jax.lax.broadcasted_iota(
                jnp.int32, (nblk_max, e), 0).astype(jnp.float32)
            be = jnp.sum((cnb_b <= jrow).astype(jnp.int32), axis=1)
            be_last = jnp.sum((cnb_b <= (nb_total - 1.0)).astype(jnp.int32),
                              axis=1)
            be_ref[...] = jnp.minimum(be, be_last)
            c0r_ref[...] = jnp.zeros_like(c0r_ref)
            c1r_ref[...] = jnp.zeros_like(c1r_ref)

        @pl.when(ph == 1)
        def _():
            tril = (jax.lax.broadcasted_iota(jnp.int32, (_RB, _RB), 0)
                    >= jax.lax.broadcasted_iota(jnp.int32, (_RB, _RB), 1)
                    ).astype(jnp.bfloat16)
            cum1 = jnp.dot(tril, oh1.astype(jnp.bfloat16),
                           preferred_element_type=jnp.float32) + c0r_ref[...]
            cum2 = jnp.dot(tril, oh2.astype(jnp.bfloat16),
                           preferred_element_type=jnp.float32) + c1r_ref[...]
            base = base_ref[...]
            c0f = c0f_ref[...]
            p0 = jnp.sum(jnp.where(oh1, base + cum1 - 1.0, 0.0), axis=1)
            p1 = jnp.sum(jnp.where(oh2, base + c0f + cum2 - 1.0, 0.0), axis=1)
            p0_ref[...] = p0.astype(jnp.int32)
            p1_ref[...] = p1.astype(jnp.int32)
            w1v = jax.nn.sigmoid(m1 - m2)[:, 0]
            w1_ref[...] = w1v
            w2_ref[...] = 1.0 - w1v
            c0r_ref[...] += s1
            c1r_ref[...] += s2

    return body


def _ffn_body(be_ref, nb_ref, x_ref, wg_ref, wu_ref, wd_ref, y_ref):
    del be_ref

    @pl.when(pl.program_id(0) < nb_ref[0])
    def _():
        x = _unpack_pair(x_ref[...])
        g = jnp.dot(x, wg_ref[0].astype(jnp.bfloat16),
                    preferred_element_type=jnp.float32)
        u = jnp.dot(x, wu_ref[0].astype(jnp.bfloat16),
                    preferred_element_type=jnp.float32)
        h = (g * jax.nn.sigmoid(g)) * u
        y = jnp.dot(h.astype(jnp.bfloat16), wd_ref[0].astype(jnp.bfloat16),
                    preferred_element_type=jnp.float32)
        dh = y.shape[1] // 2
        y_ref[...] = _pack_pair(y[:, :dh].astype(jnp.bfloat16),
                                y[:, dh:].astype(jnp.bfloat16))


def _combine_body(y0_ref, y1_ref, w0_ref, w1_ref, out_ref):
    y0 = _unpack_pair(y0_ref[...]).astype(jnp.float32)
    y1 = _unpack_pair(y1_ref[...]).astype(jnp.float32)
    out_ref[...] = w0_ref[...] * y0 + w1_ref[...] * y1


def kernel(hidden_states, Wg, W_gate, W_up, W_down):
    b, s, d = hidden_states.shape
    e, _, dff = W_gate.shape
    t = b * s
    x = hidden_states.reshape(t, d)

    nblk_max = t * _TOPK // _BM + e
    rpad = nblk_max * _BM
    dp = d // 2

    (router_logits, p0, p1, w1, w2, block_expert, nb_total) = pl.pallas_call(
        _make_routing_body(e, nblk_max),
        grid=(2, t // _RB),
        in_specs=[
            pl.BlockSpec((_RB, d), lambda ph, i: ((1 - ph) * i, 0)),
            pl.BlockSpec((d, e), lambda ph, i: (0, 0)),
        ],
        out_specs=[
            pl.BlockSpec((_RB, e), lambda ph, i: (i, 0)),
            pl.BlockSpec((_RB,), lambda ph, i: (i,)),
            pl.BlockSpec((_RB,), lambda ph, i: (i,)),
            pl.BlockSpec((_RB,), lambda ph, i: (i,)),
            pl.BlockSpec((_RB,), lambda ph, i: (i,)),
            pl.BlockSpec((nblk_max,), lambda ph, i: (0,)),
            pl.BlockSpec(memory_space=pltpu.SMEM),
        ],
        out_shape=[
            jax.ShapeDtypeStruct((t, e), jnp.float32),
            jax.ShapeDtypeStruct((t,), jnp.int32),
            jax.ShapeDtypeStruct((t,), jnp.int32),
            jax.ShapeDtypeStruct((t,), jnp.float32),
            jax.ShapeDtypeStruct((t,), jnp.float32),
            jax.ShapeDtypeStruct((nblk_max,), jnp.int32),
            jax.ShapeDtypeStruct((1,), jnp.int32),
        ],
        scratch_shapes=[
            pltpu.VMEM((1, e), jnp.float32),
            pltpu.VMEM((1, e), jnp.float32),
            pltpu.VMEM((1, e), jnp.float32),
            pltpu.VMEM((1, e), jnp.float32),
            pltpu.VMEM((t, e), jnp.float32),
        ],
    )(x, Wg)

    t_per_w = t // _NW
    mesh = plsc.VectorSubcoreMesh(core_axis_name="c", subcore_axis_name="s")

    @functools.partial(
        pl.kernel, mesh=mesh,
        out_type=jax.ShapeDtypeStruct((rpad, dp), jnp.int32),
        scratch_types=[
            pltpu.VMEM((t_per_w,), jnp.int32),
            pltpu.VMEM((t_per_w, dp), jnp.int32),
            pltpu.SemaphoreType.DMA,
        ],
    )
    def _dispatch(x_hbm, p0_hbm, p1_hbm, xd_hbm, slot_v, rows_v, sem):
        wid = lax.axis_index("s") * _NC + lax.axis_index("c")
        off = wid * t_per_w
        pltpu.sync_copy(x_hbm.at[pl.ds(off, t_per_w)], rows_v)
        pltpu.sync_copy(p0_hbm.at[pl.ds(off, t_per_w)], slot_v)
        pltpu.async_copy(rows_v, xd_hbm.at[slot_v], sem).wait()
        pltpu.sync_copy(p1_hbm.at[pl.ds(off, t_per_w)], slot_v)
        pltpu.async_copy(rows_v, xd_hbm.at[slot_v], sem).wait()

    xb = x.astype(jnp.bfloat16)
    x_packed = _pack_pair(xb[:, :dp], xb[:, dp:])
    x_disp = _dispatch(x_packed, p0, p1)

    grid_spec = pltpu.PrefetchScalarGridSpec(
        num_scalar_prefetch=2,
        grid=(nblk_max,),
        in_specs=[
            pl.BlockSpec((_BM, dp), lambda i, be, nb: (jnp.minimum(i, nb[0]), 0)),
            pl.BlockSpec((1, d, dff), lambda i, be, nb: (be[i], 0, 0)),
            pl.BlockSpec((1, d, dff), lambda i, be, nb: (be[i], 0, 0)),
            pl.BlockSpec((1, dff, d), lambda i, be, nb: (be[i], 0, 0)),
        ],
        out_specs=pl.BlockSpec((_BM, dp), lambda i, be, nb: (jnp.minimum(i, nb[0]), 0)),
    )
    y = pl.pallas_call(
        _ffn_body,
        grid_spec=grid_spec,
        out_shape=jax.ShapeDtypeStruct((rpad, dp), jnp.int32),
    )(block_expert, nb_total, x_disp, W_gate, W_up, W_down)

    g_per_w = t // _NW

    @functools.partial(
        pl.kernel, mesh=mesh,
        out_type=[jax.ShapeDtypeStruct((t, dp), jnp.int32),
                  jax.ShapeDtypeStruct((t, dp), jnp.int32)],
        scratch_types=[
            pltpu.VMEM((g_per_w,), jnp.int32),
            pltpu.VMEM((g_per_w, dp), jnp.int32),
            pltpu.SemaphoreType.DMA,
        ],
    )
    def _cgather(y_hbm, p0_hbm, p1_hbm, y0_hbm, y1_hbm, idx_v, rows_v, sem):
        wid = lax.axis_index("s") * _NC + lax.axis_index("c")
        off = wid * g_per_w
        pltpu.sync_copy(p0_hbm.at[pl.ds(off, g_per_w)], idx_v)
        pltpu.async_copy(y_hbm.at[idx_v], rows_v, sem).wait()
        pltpu.sync_copy(rows_v, y0_hbm.at[pl.ds(off, g_per_w)])
        pltpu.sync_copy(p1_hbm.at[pl.ds(off, g_per_w)], idx_v)
        pltpu.async_copy(y_hbm.at[idx_v], rows_v, sem).wait()
        pltpu.sync_copy(rows_v, y1_hbm.at[pl.ds(off, g_per_w)])

    yg0, yg1 = _cgather(y, p0, p1)

    out = pl.pallas_call(
        _combine_body,
        grid=(t // _RB,),
        in_specs=[
            pl.BlockSpec((_RB, dp), lambda i: (i, 0)),
            pl.BlockSpec((_RB, dp), lambda i: (i, 0)),
            pl.BlockSpec((_RB, 1), lambda i: (i, 0)),
            pl.BlockSpec((_RB, 1), lambda i: (i, 0)),
        ],
        out_specs=pl.BlockSpec((_RB, d), lambda i: (i, 0)),
        out_shape=jax.ShapeDtypeStruct((t, d), jnp.float32),
    )(yg0, yg1, w1.reshape(t, 1), w2.reshape(t, 1))

    return out.reshape(b, s, d), router_logits


# speedup vs baseline: 7.9728x; 1.0224x over previous
import functools

import jax
import jax.numpy as jnp
from jax import lax
from jax.experimental import pallas as pl
from jax.experimental.pallas import tpu as pltpu
from jax.experimental.pallas import tpu_sc as plsc

_TOPK = 2
_BM = 128
_RB = 1024
_NC = 2
_NS = 16
_NW = _NC * _NS


def _top2(l, e):
    rb = l.shape[0]
    lane = jax.lax.broadcasted_iota(jnp.int32, (rb, e), 1)
    m1 = jnp.max(l, axis=1, keepdims=True)
    a1 = jnp.min(jnp.where(l == m1, lane, e), axis=1, keepdims=True)
    l2 = jnp.where(lane == a1, -jnp.inf, l)
    m2 = jnp.max(l2, axis=1, keepdims=True)
    a2 = jnp.min(jnp.where(l2 == m2, lane, e), axis=1, keepdims=True)
    return a1, m1, a2, m2


def _pack_pair(lo_bf, hi_bf):
    lo = jax.lax.bitcast_convert_type(lo_bf, jnp.uint16).astype(jnp.uint32)
    hi = jax.lax.bitcast_convert_type(hi_bf, jnp.uint16).astype(jnp.uint32)
    return jax.lax.bitcast_convert_type(lo | (hi << 16), jnp.int32)


def _unpack_pair(w_i32):
    w = jax.lax.bitcast_convert_type(w_i32, jnp.uint32)
    lo = jax.lax.bitcast_convert_type((w & 0xFFFF).astype(jnp.uint16),
                                      jnp.bfloat16)
    hi = jax.lax.bitcast_convert_type((w >> 16).astype(jnp.uint16),
                                      jnp.bfloat16)
    return jnp.concatenate([lo, hi], axis=1)


def _make_routing_body(e, nblk_max):
    def body(x_ref, wg_ref, l_ref, p0_ref, p1_ref, w1_ref, w2_ref, be_ref,
             nb_ref, c0r_ref, c1r_ref, c0f_ref, base_ref, lsc_ref):
        ph = pl.program_id(0)
        i = pl.program_id(1)
        rb = x_ref.shape[0]

        @pl.when(ph == 0)
        def _():
            lsc_ref[pl.ds(i * rb, rb), :] = jnp.dot(
                x_ref[...], wg_ref[...], preferred_element_type=jnp.float32)

        l = lsc_ref[pl.ds(i * rb, rb), :]
        l_ref[...] = l
        a1, m1, a2, m2 = _top2(l, e)
        lane = jax.lax.broadcasted_iota(jnp.int32, (_RB, e), 1)
        oh1 = (lane == a1)
        oh2 = (lane == a2)
        s1 = jnp.sum(oh1.astype(jnp.float32), axis=0, keepdims=True)
        s2 = jnp.sum(oh2.astype(jnp.float32), axis=0, keepdims=True)

        @pl.when((ph == 0) & (i == 0))
        def _():
            c0r_ref[...] = jnp.zeros_like(c0r_ref)
            c1r_ref[...] = jnp.zeros_like(c1r_ref)

        @pl.when(ph == 0)
        def _():
            c0r_ref[...] += s1
            c1r_ref[...] += s2

        @pl.when((ph == 1) & (i == 0))
        def _():
            c0 = c0r_ref[...]
            counts = c0 + c1r_ref[...]
            c0f_ref[...] = c0
            tril_e = (jax.lax.broadcasted_iota(jnp.int32, (e, e), 0)
                      <= jax.lax.broadcasted_iota(jnp.int32, (e, e), 1)
                      ).astype(jnp.float32)
            nblk = jnp.ceil(counts / _BM)
            cnb = jnp.dot(nblk, tril_e, preferred_element_type=jnp.float32)
            base_ref[...] = (cnb - nblk) * _BM
            nb_total = cnb[0, e - 1]
            nb_ref[0] = nb_total.astype(jnp.int32)
            cnb_b = jnp.broadcast_to(cnb, (nblk_max, e))
            jrow = jax.lax.broadcasted_iota(
                jnp.int32, (nblk_max, e), 0).astype(jnp.float32)
            be = jnp.sum((cnb_b <= jrow).astype(jnp.int32), axis=1)
            be_last = jnp.sum((cnb_b <= (nb_total - 1.0)).astype(jnp.int32),
                              axis=1)
            be_ref[...] = jnp.minimum(be, be_last)
            c0r_ref[...] = jnp.zeros_like(c0r_ref)
            c1r_ref[...] = jnp.zeros_like(c1r_ref)

        @pl.when(ph == 1)
        def _():
            tril = (jax.lax.broadcasted_iota(jnp.int32, (_RB, _RB), 0)
                    >= jax.lax.broadcasted_iota(jnp.int32, (_RB, _RB), 1)
                    ).astype(jnp.bfloat16)
            cum1 = jnp.dot(tril, oh1.astype(jnp.bfloat16),
                           preferred_element_type=jnp.float32) + c0r_ref[...]
            cum2 = jnp.dot(tril, oh2.astype(jnp.bfloat16),
                           preferred_element_type=jnp.float32) + c1r_ref[...]
            base = base_ref[...]
            c0f = c0f_ref[...]
            p0 = jnp.sum(jnp.where(oh1, base + cum1 - 1.0, 0.0), axis=1)
            p1 = jnp.sum(jnp.where(oh2, base + c0f + cum2 - 1.0, 0.0), axis=1)
            p0_ref[...] = p0.astype(jnp.int32)
            p1_ref[...] = p1.astype(jnp.int32)
            w1v = jax.nn.sigmoid(m1 - m2)[:, 0]
            w1_ref[...] = w1v
            w2_ref[...] = 1.0 - w1v
            c0r_ref[...] += s1
            c1r_ref[...] += s2

    return body


def _ffn_body(be_ref, nb_ref, x_ref, wg_ref, wu_ref, wd_ref, y_ref):
    del be_ref

    @pl.when(pl.program_id(0) < nb_ref[0])
    def _():
        x = _unpack_pair(x_ref[...])
        g = jnp.dot(x, wg_ref[0].astype(jnp.bfloat16),
                    preferred_element_type=jnp.float32)
        u = jnp.dot(x, wu_ref[0].astype(jnp.bfloat16),
                    preferred_element_type=jnp.float32)
        h = (g * jax.nn.sigmoid(g)) * u
        y = jnp.dot(h.astype(jnp.bfloat16), wd_ref[0].astype(jnp.bfloat16),
                    preferred_element_type=jnp.float32)
        dh = y.shape[1] // 2
        y_ref[...] = _pack_pair(y[:, :dh].astype(jnp.bfloat16),
                                y[:, dh:].astype(jnp.bfloat16))


def _combine_body(y0_ref, y1_ref, w0_ref, w1_ref, out_ref):
    y0 = _unpack_pair(y0_ref[...]).astype(jnp.float32)
    y1 = _unpack_pair(y1_ref[...]).astype(jnp.float32)
    out_ref[...] = w0_ref[...] * y0 + w1_ref[...] * y1


def kernel(hidden_states, Wg, W_gate, W_up, W_down):
    b, s, d = hidden_states.shape
    e, _, dff = W_gate.shape
    t = b * s
    x = hidden_states.reshape(t, d)

    nblk_max = t * _TOPK // _BM + e
    rpad = nblk_max * _BM
    dp = d // 2

    (router_logits, p0, p1, w1, w2, block_expert, nb_total) = pl.pallas_call(
        _make_routing_body(e, nblk_max),
        grid=(2, t // _RB),
        in_specs=[
            pl.BlockSpec((_RB, d), lambda ph, i: ((1 - ph) * i, 0)),
            pl.BlockSpec((d, e), lambda ph, i: (0, 0)),
        ],
        out_specs=[
            pl.BlockSpec((_RB, e), lambda ph, i: (i, 0)),
            pl.BlockSpec((_RB,), lambda ph, i: (i,)),
            pl.BlockSpec((_RB,), lambda ph, i: (i,)),
            pl.BlockSpec((_RB,), lambda ph, i: (i,)),
            pl.BlockSpec((_RB,), lambda ph, i: (i,)),
            pl.BlockSpec((nblk_max,), lambda ph, i: (0,)),
            pl.BlockSpec(memory_space=pltpu.SMEM),
        ],
        out_shape=[
            jax.ShapeDtypeStruct((t, e), jnp.float32),
            jax.ShapeDtypeStruct((t,), jnp.int32),
            jax.ShapeDtypeStruct((t,), jnp.int32),
            jax.ShapeDtypeStruct((t,), jnp.float32),
            jax.ShapeDtypeStruct((t,), jnp.float32),
            jax.ShapeDtypeStruct((nblk_max,), jnp.int32),
            jax.ShapeDtypeStruct((1,), jnp.int32),
        ],
        scratch_shapes=[
            pltpu.VMEM((1, e), jnp.float32),
            pltpu.VMEM((1, e), jnp.float32),
            pltpu.VMEM((1, e), jnp.float32),
            pltpu.VMEM((1, e), jnp.float32),
            pltpu.VMEM((t, e), jnp.float32),
        ],
    )(x, Wg)

    t_per_w = t // _NW
    mesh = plsc.VectorSubcoreMesh(core_axis_name="c", subcore_axis_name="s")

    @functools.partial(
        pl.kernel, mesh=mesh,
        out_type=jax.ShapeDtypeStruct((rpad, dp), jnp.int32),
        scratch_types=[
            pltpu.VMEM((t_per_w,), jnp.int32),
            pltpu.VMEM((t_per_w, dp), jnp.int32),
            pltpu.SemaphoreType.DMA,
        ],
    )
    def _dispatch(x_hbm, p0_hbm, p1_hbm, xd_hbm, slot_v, rows_v, sem):
        wid = lax.axis_index("s") * _NC + lax.axis_index("c")
        off = wid * t_per_w
        pltpu.sync_copy(x_hbm.at[pl.ds(off, t_per_w)], rows_v)
        pltpu.sync_copy(p0_hbm.at[pl.ds(off, t_per_w)], slot_v)
        pltpu.async_copy(rows_v, xd_hbm.at[slot_v], sem).wait()
        pltpu.sync_copy(p1_hbm.at[pl.ds(off, t_per_w)], slot_v)
        pltpu.async_copy(rows_v, xd_hbm.at[slot_v], sem).wait()

    xb = x.astype(jnp.bfloat16)
    x_packed = _pack_pair(xb[:, :dp], xb[:, dp:])
    x_disp = _dispatch(x_packed, p0, p1)

    grid_spec = pltpu.PrefetchScalarGridSpec(
        num_scalar_prefetch=2,
        grid=(nblk_max,),
        in_specs=[
            pl.BlockSpec((_BM, dp), lambda i, be, nb: (jnp.minimum(i, nb[0]), 0)),
            pl.BlockSpec((1, d, dff), lambda i, be, nb: (be[i], 0, 0)),
            pl.BlockSpec((1, d, dff), lambda i, be, nb: (be[i], 0, 0)),
            pl.BlockSpec((1, dff, d), lambda i, be, nb: (be[i], 0, 0)),
        ],
        out_specs=pl.BlockSpec((_BM, dp), lambda i, be, nb: (jnp.minimum(i, nb[0]), 0)),
    )
    y = pl.pallas_call(
        _ffn_body,
        grid_spec=grid_spec,
        out_shape=jax.ShapeDtypeStruct((rpad, dp), jnp.int32),
    )(block_expert, nb_total, x_disp, W_gate, W_up, W_down)

    g_per_w = t // _NW

    @functools.partial(
        pl.kernel, mesh=mesh,
        out_type=[jax.ShapeDtypeStruct((t, dp), jnp.int32),
                  jax.ShapeDtypeStruct((t, dp), jnp.int32)],
        scratch_types=[
            pltpu.VMEM((g_per_w,), jnp.int32),
            pltpu.VMEM((g_per_w, dp), jnp.int32),
            pltpu.SemaphoreType.DMA,
        ],
    )
    def _cgather(y_hbm, p0_hbm, p1_hbm, y0_hbm, y1_hbm, idx_v, rows_v, sem):
        wid = lax.axis_index("s") * _NC + lax.axis_index("c")
        off = wid * g_per_w
        pltpu.sync_copy(p0_hbm.at[pl.ds(off, g_per_w)], idx_v)
        pltpu.async_copy(y_hbm.at[idx_v], rows_v, sem).wait()
        pltpu.sync_copy(rows_v, y0_hbm.at[pl.ds(off, g_per_w)])
        pltpu.sync_copy(p1_hbm.at[pl.ds(off, g_per_w)], idx_v)
        pltpu.async_copy(y_hbm.at[idx_v], rows_v, sem).wait()
        pltpu.sync_copy(rows_v, y1_hbm.at[pl.ds(off, g_per_w)])

    yg0, yg1 = _cgather(y, p0, p1)

    out = pl.pallas_call(
        _combine_body,
        grid=(t // _RB,),
        in_specs=[
            pl.BlockSpec((_RB, dp), lambda i: (i, 0)),
            pl.BlockSpec((_RB, dp), lambda i: (i, 0)),
            pl.BlockSpec((_RB, 1), lambda i: (i, 0)),
            pl.BlockSpec((_RB, 1), lambda i: (i, 0)),
        ],
        out_specs=pl.BlockSpec((_RB, d), lambda i: (i, 0)),
        out_shape=jax.ShapeDtypeStruct((t, d), jnp.float32),
    )(yg0, yg1, w1.reshape(t, 1), w2.reshape(t, 1))

    return out.reshape(b, s, d), router_logits
